# baseline (speedup 1.0000x reference)
import functools

import jax
import jax.numpy as jnp
from jax import lax
from jax.experimental import pallas as pl
from jax.experimental.pallas import tpu as pltpu
from jax.experimental.pallas import tpu_sc as plsc

_B = 4096
_F = 26
_K = 64
_FK = _F * _K
_CT = _FK // 128
_NN0 = 1024
_NN1 = 512
_EPS = 1e-5

_NC = 2
_NS = 16
_NW = _NC * _NS
_BPW = _B // _NW
_RPW = _BPW * _F
_SLOT = 128
_NSTEP = _RPW // _SLOT


def _sc_gather(xp, xt, emb1, emb2):
    mesh = plsc.VectorSubcoreMesh(core_axis_name="c", subcore_axis_name="s")

    @functools.partial(
        pl.kernel,
        out_type=[
            jax.ShapeDtypeStruct((_CT, 2 * _B, _K), jnp.bfloat16),
            jax.ShapeDtypeStruct((_B,), jnp.float32),
        ],
        mesh=mesh,
        compiler_params=pltpu.CompilerParams(use_tc_tiling_on_sc=False),
        scratch_types=[
            pltpu.VMEM((_NSTEP, _SLOT), jnp.int32),
            pltpu.VMEM((_F, _BPW), jnp.int32),
            pltpu.VMEM((_SLOT, _K), jnp.bfloat16),
            pltpu.VMEM((_SLOT, _K), jnp.bfloat16),
            pltpu.VMEM((_F, _BPW), jnp.float32),
            pltpu.VMEM((_BPW,), jnp.float32),
            pltpu.SemaphoreType.DMA,
            pltpu.SemaphoreType.DMA,
            pltpu.SemaphoreType.DMA,
            pltpu.SemaphoreType.DMA,
            pltpu.SemaphoreType.DMA,
        ],
    )
    def gather_k(xp_hbm, xt_hbm, emb1_hbm, emb2_hbm, nn_hbm, fm1_hbm,
                 idx_v, idx2_v, slot0, slot1, e1_v, fm1_v,
                 s_in0, s_in1, s_out0, s_out1, s_e1):
        w = lax.axis_index("s") * _NC + lax.axis_index("c")
        pltpu.sync_copy(xp_hbm.at[w], idx_v)
        pltpu.sync_copy(xt_hbm.at[w], idx2_v)

        def e1_fire(j, carry):
            pltpu.async_copy(emb1_hbm.at[idx2_v.at[j]], e1_v.at[j], s_e1)
            return carry

        lax.fori_loop(0, _F, e1_fire, 0)

        slots = (slot0, slot1)
        sin = (s_in0, s_in1)
        sout = (s_out0, s_out1)

        def nn_dst(j):
            c = j // 2
            return nn_hbm.at[c, pl.ds(w * 2 * _SLOT + (j % 2) * _SLOT, _SLOT)]

        for b in range(2):
            pltpu.async_copy(emb2_hbm.at[idx_v.at[b]], slots[b], sin[b])

        def step(g, carry):
            for b in range(2):
                j = g * 2 + b
                pltpu.make_async_copy(emb2_hbm.at[idx_v.at[j]], slots[b], sin[b]).wait()
                pltpu.async_copy(slots[b], nn_dst(j), sout[b])
                pltpu.make_async_copy(slots[b], nn_dst(j), sout[b]).wait()
                pltpu.async_copy(emb2_hbm.at[idx_v.at[j + 2]], slots[b], sin[b])
            return carry

        lax.fori_loop(0, _NSTEP // 2 - 1, step, 0)

        for b in range(2):
            j = _NSTEP - 2 + b
            pltpu.make_async_copy(emb2_hbm.at[idx_v.at[j]], slots[b], sin[b]).wait()
            pltpu.async_copy(slots[b], nn_dst(j), sout[b])
            pltpu.make_async_copy(slots[b], nn_dst(j), sout[b]).wait()

        def e1_drain(j, carry):
            pltpu.make_async_copy(emb1_hbm.at[idx2_v.at[j]], e1_v.at[j], s_e1).wait()
            return carry

        lax.fori_loop(0, _F, e1_drain, 0)

        for g in range(_BPW // 16):
            acc = e1_v[0, pl.ds(g * 16, 16)]
            for f in range(1, _F):
                acc = acc + e1_v[f, pl.ds(g * 16, 16)]
            fm1_v[pl.ds(g * 16, 16)] = acc
        pltpu.sync_copy(fm1_v, fm1_hbm.at[pl.ds(w * _BPW, _BPW)])

    return gather_k(xp, xt, emb1, emb2)


def _tc_mlp(nn3, fm1, bias, g1, b1, be1, g2, b2, be2, W1, W2):
    BT = 512
    grid = (_B // BT,)
    inv = float((1.0 + _EPS) ** -0.5)

    def mlp_k(nn_ref, fm1_ref, bias_ref, g1_ref, b1_ref, be1_ref,
              g2_ref, b2_ref, be2_ref, W1_ref, W2_ref, out_ref):
        fold = None
        sq = None
        acc = None
        for c in range(_CT):
            slab = nn_ref[c]
            slab_f = slab.astype(jnp.float32)
            fold = slab_f if fold is None else fold + slab_f
            s = jnp.sum(slab_f * slab_f, axis=1)
            sq = s if sq is None else sq + s
            p = lax.dot_general(slab, W1_ref[:, 128 * c:128 * (c + 1)],
                                (((1,), (1,)), ((), ())),
                                preferred_element_type=jnp.float32)
            acc = p if acc is None else acc + p
        sum_f = (lax.slice_in_dim(fold, 0, _K, axis=1)
                 + lax.slice_in_dim(fold, _K, 2 * _K, axis=1))
        fm2 = 0.5 * (jnp.sum(sum_f * sum_f, axis=1) - sq)
        a1 = g1_ref[...] * inv
        c1 = b1_ref[...] * a1 + be1_ref[...]
        h = jnp.maximum(acc * a1 + c1, 0.0)
        a2 = g2_ref[...] * inv
        c2 = b2_ref[...] * a2 + be2_ref[...]
        h = lax.dot_general(h.astype(jnp.bfloat16), W2_ref[...],
                            (((1,), (1,)), ((), ())),
                            preferred_element_type=jnp.float32)
        h = jnp.maximum(h * a2 + c2, 0.0)
        out_ref[...] = fm1_ref[...] + fm2 + jnp.sum(h, axis=1) + bias_ref[0]

    return pl.pallas_call(
        mlp_k,
        grid=grid,
        in_specs=[
            pl.BlockSpec((_CT, BT, 128), lambda i: (0, i, 0)),
            pl.BlockSpec((BT,), lambda i: (i,)),
            pl.BlockSpec(memory_space=pltpu.SMEM),
            pl.BlockSpec((1, _NN0), lambda i: (0, 0)),
            pl.BlockSpec((1, _NN0), lambda i: (0, 0)),
            pl.BlockSpec((1, _NN0), lambda i: (0, 0)),
            pl.BlockSpec((1, _NN1), lambda i: (0, 0)),
            pl.BlockSpec((1, _NN1), lambda i: (0, 0)),
            pl.BlockSpec((1, _NN1), lambda i: (0, 0)),
            pl.BlockSpec((_NN0, _FK), lambda i: (0, 0)),
            pl.BlockSpec((_NN1, _NN0), lambda i: (0, 0)),
        ],
        out_specs=pl.BlockSpec((BT,), lambda i: (i,)),
        out_shape=jax.ShapeDtypeStruct((_B,), jnp.float32),
        compiler_params=pltpu.CompilerParams(
            dimension_semantics=("arbitrary",),
        ),
    )(nn3, fm1, bias, g1, b1, be1, g2, b2, be2, W1, W2)


def kernel(x, bias, emb1, emb2, W1, b1, g1, be1, W2, b2, g2, be2):
    xp = (x.reshape(_NW, 2, 64, _CT, 2)
           .transpose(0, 3, 1, 2, 4)
           .reshape(_NW, _NSTEP, _SLOT))
    xt = x.reshape(_NW, _BPW, _F).transpose(0, 2, 1)
    nn, fm1 = _sc_gather(xp, xt, emb1.reshape(-1),
                         emb2.astype(jnp.bfloat16))
    nn3 = nn.reshape(_CT, _B, 128)
    return _tc_mlp(
        nn3, fm1, bias,
        g1.reshape(1, -1), b1.reshape(1, -1), be1.reshape(1, -1),
        g2.reshape(1, -1), b2.reshape(1, -1), be2.reshape(1, -1),
        W1.astype(jnp.bfloat16), W2.astype(jnp.bfloat16),
    )


# speedup vs baseline: 1.7632x; 1.7632x over previous
import functools

import jax
import jax.numpy as jnp
from jax import lax
from jax.experimental import pallas as pl
from jax.experimental.pallas import tpu as pltpu
from jax.experimental.pallas import tpu_sc as plsc

_B = 4096
_F = 26
_K = 64
_FK = _F * _K
_CT = _FK // 128
_NN0 = 1024
_NN1 = 512
_EPS = 1e-5

_NC = 2
_NS = 16
_NW = _NC * _NS
_SLOT = 128
_SPLIT = 2
_BH = _B // _SPLIT
_BPW = _BH // _NW
_NSTEP = _BPW * _F // _SLOT
_SPC = _NSTEP // _CT


def _sc_gather(xp, xt, emb1, emb2):
    mesh = plsc.VectorSubcoreMesh(core_axis_name="c", subcore_axis_name="s")

    @functools.partial(
        pl.kernel,
        out_type=[
            jax.ShapeDtypeStruct((_CT, 2 * _BH, _K), jnp.float32),
            jax.ShapeDtypeStruct((_BH,), jnp.float32),
        ],
        mesh=mesh,
        compiler_params=pltpu.CompilerParams(use_tc_tiling_on_sc=False),
        scratch_types=[
            pltpu.VMEM((_NSTEP, _SLOT), jnp.int32),
            pltpu.VMEM((_F, _BPW), jnp.int32),
            pltpu.VMEM((_SLOT, _K), jnp.float32),
            pltpu.VMEM((_SLOT, _K), jnp.float32),
            pltpu.VMEM((_F, _BPW), jnp.float32),
            pltpu.VMEM((_BPW,), jnp.float32),
            pltpu.SemaphoreType.DMA,
            pltpu.SemaphoreType.DMA,
            pltpu.SemaphoreType.DMA,
            pltpu.SemaphoreType.DMA,
            pltpu.SemaphoreType.DMA,
        ],
    )
    def gather_k(xp_hbm, xt_hbm, emb1_hbm, emb2_hbm, nn_hbm, fm1_hbm,
                 idx_v, idx2_v, slot0, slot1, e1_v, fm1_v,
                 s_in0, s_in1, s_out0, s_out1, s_e1):
        w = lax.axis_index("s") * _NC + lax.axis_index("c")
        pltpu.sync_copy(xp_hbm.at[w], idx_v)
        pltpu.sync_copy(xt_hbm.at[w], idx2_v)

        def e1_fire(j, carry):
            pltpu.async_copy(emb1_hbm.at[idx2_v.at[j]], e1_v.at[j], s_e1)
            return carry

        lax.fori_loop(0, _F, e1_fire, 0)

        slots = (slot0, slot1)
        sin = (s_in0, s_in1)
        sout = (s_out0, s_out1)

        def nn_dst(j):
            c = j // _SPC
            return nn_hbm.at[c, pl.ds(w * _SPC * _SLOT + (j % _SPC) * _SLOT, _SLOT)]

        for b in range(2):
            pltpu.async_copy(emb2_hbm.at[idx_v.at[b]], slots[b], sin[b])

        G = (_NSTEP - 2) // 2

        def step(g, carry):
            for b in range(2):
                j = g * 2 + b
                pltpu.make_async_copy(emb2_hbm.at[idx_v.at[j]], slots[b], sin[b]).wait()
                pltpu.async_copy(slots[b], nn_dst(j), sout[b])
                pltpu.make_async_copy(slots[b], nn_dst(j), sout[b]).wait()
                pltpu.async_copy(emb2_hbm.at[idx_v.at[j + 2]], slots[b], sin[b])
            return carry

        lax.fori_loop(0, G, step, 0)

        for j in range(2 * G, _NSTEP):
            b = j % 2
            pltpu.make_async_copy(emb2_hbm.at[idx_v.at[j]], slots[b], sin[b]).wait()
            pltpu.async_copy(slots[b], nn_dst(j), sout[b])
            pltpu.make_async_copy(slots[b], nn_dst(j), sout[b]).wait()
            if j + 2 < _NSTEP:
                pltpu.async_copy(emb2_hbm.at[idx_v.at[j + 2]], slots[b], sin[b])

        def e1_drain(j, carry):
            pltpu.make_async_copy(emb1_hbm.at[idx2_v.at[j]], e1_v.at[j], s_e1).wait()
            return carry

        lax.fori_loop(0, _F, e1_drain, 0)

        for g in range(_BPW // 16):
            acc = e1_v[0, pl.ds(g * 16, 16)]
            for f in range(1, _F):
                acc = acc + e1_v[f, pl.ds(g * 16, 16)]
            fm1_v[pl.ds(g * 16, 16)] = acc
        pltpu.sync_copy(fm1_v, fm1_hbm.at[pl.ds(w * _BPW, _BPW)])

    return gather_k(xp, xt, emb1, emb2)


def _tc_mlp(nn3, fm1, bias, g1, b1, be1, g2, b2, be2, W1b, W2b):
    BT = 512
    grid = (_BH // BT,)
    inv = float((1.0 + _EPS) ** -0.5)

    def mlp_k(nn_ref, fm1_ref, bias_ref, g1_ref, b1_ref, be1_ref,
              g2_ref, b2_ref, be2_ref, W1_ref, W2_ref, out_ref):
        fold = None
        sq = None
        acc = None
        for c in range(_CT):
            slab = nn_ref[c]
            fold = slab if fold is None else fold + slab
            s = jnp.sum(slab * slab, axis=1)
            sq = s if sq is None else sq + s
            p = lax.dot_general(slab.astype(jnp.bfloat16),
                                W1_ref[:, 128 * c:128 * (c + 1)],
                                (((1,), (1,)), ((), ())),
                                preferred_element_type=jnp.float32)
            acc = p if acc is None else acc + p
        sum_f = (lax.slice_in_dim(fold, 0, _K, axis=1)
                 + lax.slice_in_dim(fold, _K, 2 * _K, axis=1))
        fm2 = 0.5 * (jnp.sum(sum_f * sum_f, axis=1) - sq)
        a1 = g1_ref[...] * inv
        c1 = b1_ref[...] * a1 + be1_ref[...]
        h = jnp.maximum(acc * a1 + c1, 0.0)
        a2 = g2_ref[...] * inv
        c2 = b2_ref[...] * a2 + be2_ref[...]
        h = lax.dot_general(h.astype(jnp.bfloat16), W2_ref[...],
                            (((1,), (1,)), ((), ())),
                            preferred_element_type=jnp.float32)
        h = jnp.maximum(h * a2 + c2, 0.0)
        out_ref[...] = fm1_ref[...] + fm2 + jnp.sum(h, axis=1) + bias_ref[0]

    return pl.pallas_call(
        mlp_k,
        grid=grid,
        in_specs=[
            pl.BlockSpec((_CT, BT, 128), lambda i: (0, i, 0)),
            pl.BlockSpec((BT,), lambda i: (i,)),
            pl.BlockSpec(memory_space=pltpu.SMEM),
            pl.BlockSpec((1, _NN0), lambda i: (0, 0)),
            pl.BlockSpec((1, _NN0), lambda i: (0, 0)),
            pl.BlockSpec((1, _NN0), lambda i: (0, 0)),
            pl.BlockSpec((1, _NN1), lambda i: (0, 0)),
            pl.BlockSpec((1, _NN1), lambda i: (0, 0)),
            pl.BlockSpec((1, _NN1), lambda i: (0, 0)),
            pl.BlockSpec((_NN0, _FK), lambda i: (0, 0)),
            pl.BlockSpec((_NN1, _NN0), lambda i: (0, 0)),
        ],
        out_specs=pl.BlockSpec((BT,), lambda i: (i,)),
        out_shape=jax.ShapeDtypeStruct((_BH,), jnp.float32),
        compiler_params=pltpu.CompilerParams(
            dimension_semantics=("arbitrary",),
        ),
    )(nn3, fm1, bias, g1, b1, be1, g2, b2, be2, W1b, W2b)


def kernel(x, bias, emb1, emb2, W1, b1, g1, be1, W2, b2, g2, be2):
    emb1f = emb1.reshape(-1)
    W1b = W1.astype(jnp.bfloat16)
    W2b = W2.astype(jnp.bfloat16)
    g1r, b1r, be1r = g1.reshape(1, -1), b1.reshape(1, -1), be1.reshape(1, -1)
    g2r, b2r, be2r = g2.reshape(1, -1), b2.reshape(1, -1), be2.reshape(1, -1)

    outs = []
    for h in range(_SPLIT):
        xh = lax.slice_in_dim(x, h * _BH, (h + 1) * _BH, axis=0)
        xp = (xh.reshape(_NW, _BPW, _CT, 2)
                .transpose(0, 2, 1, 3)
                .reshape(_NW, _NSTEP, _SLOT))
        xt = xh.reshape(_NW, _BPW, _F).transpose(0, 2, 1)
        nn, fm1 = _sc_gather(xp, xt, emb1f, emb2)
        nn3 = nn.reshape(_CT, _BH, 128)
        outs.append(_tc_mlp(nn3, fm1, bias,
                            g1r, b1r, be1r, g2r, b2r, be2r, W1b, W2b))
    return jnp.concatenate(outs, axis=0)


# speedup vs baseline: 1.7861x; 1.0130x over previous
import functools

import jax
import jax.numpy as jnp
from jax import lax
from jax.experimental import pallas as pl
from jax.experimental.pallas import tpu as pltpu
from jax.experimental.pallas import tpu_sc as plsc

_B = 4096
_F = 26
_K = 64
_FK = _F * _K
_CT = _FK // 128
_NN0 = 1024
_NN1 = 512
_EPS = 1e-5

_NC = 2
_NS = 16
_NW = _NC * _NS
_SLOT = 128
_SPLIT = 2
_BH = _B // _SPLIT
_BPW = _BH // _NW
_NSTEP = _BPW * _F // _SLOT
_SPC = _NSTEP // _CT


def _sc_fm1(xt, emb1):
    mesh = plsc.VectorSubcoreMesh(core_axis_name="c", subcore_axis_name="s")

    @functools.partial(
        pl.kernel,
        out_type=[jax.ShapeDtypeStruct((_BH,), jnp.float32)],
        mesh=mesh,
        compiler_params=pltpu.CompilerParams(use_tc_tiling_on_sc=False),
        scratch_types=[
            pltpu.VMEM((_F, _BPW), jnp.int32),
            pltpu.VMEM((_F, _BPW), jnp.float32),
            pltpu.VMEM((_BPW,), jnp.float32),
            pltpu.SemaphoreType.DMA,
        ],
    )
    def fm1_k(xt_hbm, emb1_hbm, fm1_hbm, idx2_v, e1_v, fm1_v, s_e1):
        w = lax.axis_index("s") * _NC + lax.axis_index("c")
        pltpu.sync_copy(xt_hbm.at[w], idx2_v)

        def e1_fire(j, carry):
            pltpu.async_copy(emb1_hbm.at[idx2_v.at[j]], e1_v.at[j], s_e1)
            return carry

        lax.fori_loop(0, _F, e1_fire, 0)

        def e1_drain(j, carry):
            pltpu.make_async_copy(emb1_hbm.at[idx2_v.at[j]], e1_v.at[j], s_e1).wait()
            return carry

        lax.fori_loop(0, _F, e1_drain, 0)

        for g in range(_BPW // 16):
            acc = e1_v[0, pl.ds(g * 16, 16)]
            for f in range(1, _F):
                acc = acc + e1_v[f, pl.ds(g * 16, 16)]
            fm1_v[pl.ds(g * 16, 16)] = acc
        pltpu.sync_copy(fm1_v, fm1_hbm.at[pl.ds(w * _BPW, _BPW)])

    return fm1_k(xt, emb1)[0]


_NBUF = 4


def _sc_gather(xp, emb2):
    mesh = plsc.VectorSubcoreMesh(core_axis_name="c", subcore_axis_name="s")

    @functools.partial(
        pl.kernel,
        out_type=[jax.ShapeDtypeStruct((_CT, 2 * _BH, _K), jnp.float32)],
        mesh=mesh,
        compiler_params=pltpu.CompilerParams(use_tc_tiling_on_sc=False),
        scratch_types=[
            pltpu.VMEM((_NSTEP, _SLOT), jnp.int32),
        ] + [pltpu.VMEM((_SLOT, _K), jnp.float32)] * _NBUF
          + [pltpu.SemaphoreType.DMA] * (2 * _NBUF),
    )
    def gather_k(xp_hbm, emb2_hbm, nn_hbm, idx_v, *bufs_sems):
        slots = bufs_sems[:_NBUF]
        sin = bufs_sems[_NBUF:2 * _NBUF]
        sout = bufs_sems[2 * _NBUF:3 * _NBUF]
        w = lax.axis_index("s") * _NC + lax.axis_index("c")
        pltpu.sync_copy(xp_hbm.at[w], idx_v)

        def nn_dst(j):
            c = j // _SPC
            return nn_hbm.at[c, pl.ds(w * _SPC * _SLOT + (j % _SPC) * _SLOT, _SLOT)]

        for b in range(_NBUF):
            pltpu.async_copy(emb2_hbm.at[idx_v.at[b]], slots[b], sin[b])

        G = (_NSTEP - _NBUF) // _NBUF

        def step(g, carry):
            for b in range(_NBUF):
                j = g * _NBUF + b
                pltpu.make_async_copy(emb2_hbm.at[idx_v.at[j]], slots[b], sin[b]).wait()
                pltpu.async_copy(slots[b], nn_dst(j), sout[b])
                pltpu.make_async_copy(slots[b], nn_dst(j), sout[b]).wait()
                pltpu.async_copy(emb2_hbm.at[idx_v.at[j + _NBUF]], slots[b], sin[b])
            return carry

        lax.fori_loop(0, G, step, 0)

        for j in range(G * _NBUF, _NSTEP):
            b = j % _NBUF
            pltpu.make_async_copy(emb2_hbm.at[idx_v.at[j]], slots[b], sin[b]).wait()
            pltpu.async_copy(slots[b], nn_dst(j), sout[b])
            pltpu.make_async_copy(slots[b], nn_dst(j), sout[b]).wait()
            if j + _NBUF < _NSTEP:
                pltpu.async_copy(emb2_hbm.at[idx_v.at[j + _NBUF]], slots[b], sin[b])

    return gather_k(xp, emb2)[0]


def _tc_mlp(nn3, fm1, bias, g1, b1, be1, g2, b2, be2, W1b, W2b):
    BT = 512
    grid = (_BH // BT,)
    inv = float((1.0 + _EPS) ** -0.5)

    def mlp_k(nn_ref, fm1_ref, bias_ref, g1_ref, b1_ref, be1_ref,
              g2_ref, b2_ref, be2_ref, W1_ref, W2_ref, out_ref):
        fold = None
        sq = None
        acc = None
        for c in range(_CT):
            slab = nn_ref[c]
            fold = slab if fold is None else fold + slab
            s = jnp.sum(slab * slab, axis=1)
            sq = s if sq is None else sq + s
            p = lax.dot_general(slab.astype(jnp.bfloat16),
                                W1_ref[:, 128 * c:128 * (c + 1)],
                                (((1,), (1,)), ((), ())),
                                preferred_element_type=jnp.float32)
            acc = p if acc is None else acc + p
        sum_f = (lax.slice_in_dim(fold, 0, _K, axis=1)
                 + lax.slice_in_dim(fold, _K, 2 * _K, axis=1))
        fm2 = 0.5 * (jnp.sum(sum_f * sum_f, axis=1) - sq)
        a1 = g1_ref[...] * inv
        c1 = b1_ref[...] * a1 + be1_ref[...]
        h = jnp.maximum(acc * a1 + c1, 0.0)
        a2 = g2_ref[...] * inv
        c2 = b2_ref[...] * a2 + be2_ref[...]
        h = lax.dot_general(h.astype(jnp.bfloat16), W2_ref[...],
                            (((1,), (1,)), ((), ())),
                            preferred_element_type=jnp.float32)
        h = jnp.maximum(h * a2 + c2, 0.0)
        out_ref[...] = fm1_ref[...] + fm2 + jnp.sum(h, axis=1) + bias_ref[0]

    return pl.pallas_call(
        mlp_k,
        grid=grid,
        in_specs=[
            pl.BlockSpec((_CT, BT, 128), lambda i: (0, i, 0)),
            pl.BlockSpec((BT,), lambda i: (i,)),
            pl.BlockSpec(memory_space=pltpu.SMEM),
            pl.BlockSpec((1, _NN0), lambda i: (0, 0)),
            pl.BlockSpec((1, _NN0), lambda i: (0, 0)),
            pl.BlockSpec((1, _NN0), lambda i: (0, 0)),
            pl.BlockSpec((1, _NN1), lambda i: (0, 0)),
            pl.BlockSpec((1, _NN1), lambda i: (0, 0)),
            pl.BlockSpec((1, _NN1), lambda i: (0, 0)),
            pl.BlockSpec((_NN0, _FK), lambda i: (0, 0)),
            pl.BlockSpec((_NN1, _NN0), lambda i: (0, 0)),
        ],
        out_specs=pl.BlockSpec((BT,), lambda i: (i,)),
        out_shape=jax.ShapeDtypeStruct((_BH,), jnp.float32),
        compiler_params=pltpu.CompilerParams(
            dimension_semantics=("arbitrary",),
        ),
    )(nn3, fm1, bias, g1, b1, be1, g2, b2, be2, W1b, W2b)


def kernel(x, bias, emb1, emb2, W1, b1, g1, be1, W2, b2, g2, be2):
    emb1f = emb1.reshape(-1)
    W1b = W1.astype(jnp.bfloat16)
    W2b = W2.astype(jnp.bfloat16)
    g1r, b1r, be1r = g1.reshape(1, -1), b1.reshape(1, -1), be1.reshape(1, -1)
    g2r, b2r, be2r = g2.reshape(1, -1), b2.reshape(1, -1), be2.reshape(1, -1)

    halves = [lax.slice_in_dim(x, h * _BH, (h + 1) * _BH, axis=0)
              for h in range(_SPLIT)]
    fm1s = [_sc_fm1(xh.reshape(_NW, _BPW, _F).transpose(0, 2, 1), emb1f)
            for xh in halves]
    outs = []
    for h, xh in enumerate(halves):
        xp = (xh.reshape(_NW, _BPW, _CT, 2)
                .transpose(0, 2, 1, 3)
                .reshape(_NW, _NSTEP, _SLOT))
        nn = _sc_gather(xp, emb2)
        nn3 = nn.reshape(_CT, _BH, 128)
        outs.append(_tc_mlp(nn3, fm1s[h], bias,
                            g1r, b1r, be1r, g2r, b2r, be2r, W1b, W2b))
    return jnp.concatenate(outs, axis=0)
